# baseline (speedup 1.0000x reference)
import functools

import jax
import jax.numpy as jnp
from jax import lax
from jax.experimental import pallas as pl
from jax.experimental.pallas import tpu as pltpu
from jax.experimental.pallas import tpu_sc as plsc

BATCH = 16384
HIDDEN = 32
NROWS = 1000000

_NC = 2
_NS = 16
_NW = _NC * _NS
_BPW = BATCH // _NW
_CHUNK = 128
_NCHUNK = _BPW // _CHUNK
_LANES = 16

_FULL_TILES = NROWS // 128
_TAIL = NROWS - _FULL_TILES * 128
_TPW = 245
_WIN = 4096


def _detile_body(uT, iT, u_tail, i_tail, u_flat, i_flat,
                 b0, b1, o0, o1, tailb, si0, si1, so0, so1):
    wid = lax.axis_index("s") * _NC + lax.axis_index("c")
    start = wid * _TPW
    lane = lax.iota(jnp.int32, _LANES)
    perm = [(lane + s) & 15 for s in range(_LANES)]
    permk32 = [lane * HIDDEN + p for p in perm]

    def tile_of(g):
        return jnp.minimum(start + g, _FULL_TILES - 1)

    def transpose_win(buf, out_v):
        def blk(kk):
            kvec = lane + kk * _LANES
            koff = kk * (_LANES * HIDDEN)
            for h0 in (0, 16):
                for s in range(_LANES):
                    hv = perm[s] + h0
                    sidx = permk32[s] + (koff + h0)
                    v = plsc.load_gather(buf, [hv, kvec])
                    plsc.store_scatter(out_v, [sidx], v)

        plsc.parallel_loop(0, 128 // _LANES, 1, unroll=2)(blk)

    for tT, flat in ((uT, u_flat), (iT, i_flat)):
        pltpu.async_copy(tT.at[:, pl.ds(tile_of(0) * 128, 128)], b0, si0)

        def pair(p, carry):
            g0 = 2 * p
            pltpu.make_async_copy(tT.at[:, pl.ds(0, 128)], b0, si0).wait()
            pltpu.async_copy(tT.at[:, pl.ds(tile_of(g0 + 1) * 128, 128)], b1, si1)
            transpose_win(b0, o0)
            pltpu.async_copy(o0, flat.at[pl.ds(tile_of(g0) * _WIN, _WIN)], so0)
            pltpu.make_async_copy(tT.at[:, pl.ds(0, 128)], b1, si1).wait()
            pltpu.async_copy(tT.at[:, pl.ds(tile_of(g0 + 2) * 128, 128)], b0, si0)
            transpose_win(b1, o1)
            pltpu.make_async_copy(flat.at[pl.ds(0, _WIN)], o0, so0).wait()
            pltpu.async_copy(o1, flat.at[pl.ds(tile_of(g0 + 1) * _WIN, _WIN)], so1)
            pltpu.make_async_copy(flat.at[pl.ds(0, _WIN)], o1, so1).wait()
            return carry

        lax.fori_loop(0, _TPW // 2 + 1, pair, 0)
        pltpu.make_async_copy(tT.at[:, pl.ds(0, 128)], b0, si0).wait()

    @pl.when(wid == 0)
    def _():
        pltpu.sync_copy(u_tail, tailb)
        pltpu.sync_copy(tailb,
                        u_flat.at[pl.ds(_FULL_TILES * _WIN, _TAIL * HIDDEN)])

    @pl.when(wid == 1)
    def _():
        pltpu.sync_copy(i_tail, tailb)
        pltpu.sync_copy(tailb,
                        i_flat.at[pl.ds(_FULL_TILES * _WIN, _TAIL * HIDDEN)])


def _detile(uT, iT, u_tail, i_tail):
    mesh = plsc.VectorSubcoreMesh(core_axis_name="c", subcore_axis_name="s")
    f = functools.partial(
        pl.kernel,
        mesh=mesh,
        compiler_params=pltpu.CompilerParams(use_tc_tiling_on_sc=True,
                                             needs_layout_passes=False),
        out_type=(
            jax.ShapeDtypeStruct((NROWS * HIDDEN,), jnp.float32),
            jax.ShapeDtypeStruct((NROWS * HIDDEN,), jnp.float32),
        ),
        scratch_types=[
            pltpu.VMEM((HIDDEN, 128), jnp.float32),
            pltpu.VMEM((HIDDEN, 128), jnp.float32),
            pltpu.VMEM((_WIN,), jnp.float32),
            pltpu.VMEM((_WIN,), jnp.float32),
            pltpu.VMEM((_TAIL * HIDDEN,), jnp.float32),
            pltpu.SemaphoreType.DMA,
            pltpu.SemaphoreType.DMA,
            pltpu.SemaphoreType.DMA,
            pltpu.SemaphoreType.DMA,
        ],
    )(_detile_body)
    return f(uT, iT, u_tail, i_tail)


def _sc_body(trip_hbm, uemb, iemb, ibias,
             x_out, l2_out,
             trip_v, idxu_v, idxi_v, idxj_v,
             u_v, i_v, j_v, ib_v, jb_v, x_v, l2_v, sem):
    wid = lax.axis_index("s") * _NC + lax.axis_index("c")
    base = wid * _BPW

    pltpu.sync_copy(trip_hbm.at[pl.ds(base, _BPW)], trip_v)

    lane = lax.iota(jnp.int32, _LANES)
    for k in range(_BPW // _LANES):
        rows = lane + (k * _LANES)
        dst = pl.ds(k * _LANES, _LANES)
        idxu_v[dst] = plsc.load_gather(trip_v, [rows, jnp.zeros((_LANES,), jnp.int32)])
        idxi_v[dst] = plsc.load_gather(trip_v, [rows, jnp.full((_LANES,), 1, jnp.int32)])
        idxj_v[dst] = plsc.load_gather(trip_v, [rows, jnp.full((_LANES,), 2, jnp.int32)])

    copies = []
    for k in range(_NCHUNK):
        s = pl.ds(k * _CHUNK, _CHUNK)
        copies.append(pltpu.async_copy(uemb.at[idxu_v.at[s]], u_v.at[s], sem))
        copies.append(pltpu.async_copy(iemb.at[idxi_v.at[s]], i_v.at[s], sem))
        copies.append(pltpu.async_copy(iemb.at[idxj_v.at[s]], j_v.at[s], sem))
        copies.append(pltpu.async_copy(ibias.at[idxi_v.at[s]], ib_v.at[s], sem))
        copies.append(pltpu.async_copy(ibias.at[idxj_v.at[s]], jb_v.at[s], sem))
    for c in copies:
        c.wait()

    def step(k, l2acc):
        off = k * _LANES
        rows = lane + off
        acc = ib_v[pl.ds(off, _LANES)] - jb_v[pl.ds(off, _LANES)]
        for h in range(HIDDEN):
            hv = jnp.full((_LANES,), h, jnp.int32)
            uh = plsc.load_gather(u_v, [rows, hv])
            ih = plsc.load_gather(i_v, [rows, hv])
            jh = plsc.load_gather(j_v, [rows, hv])
            acc = acc + uh * (ih - jh)
            l2acc = l2acc + (uh * uh + ih * ih + jh * jh)
        x_v[pl.ds(off, _LANES)] = acc
        return l2acc

    l2acc = lax.fori_loop(0, _BPW // _LANES, step, jnp.zeros((_LANES,), jnp.float32))
    l2_v[...] = l2acc

    pltpu.sync_copy(x_v, x_out.at[pl.ds(base, _BPW)])
    pltpu.sync_copy(l2_v, l2_out.at[wid])


def _sc_call(trip, user_embedding, item_embedding, item_bias):
    mesh = plsc.VectorSubcoreMesh(core_axis_name="c", subcore_axis_name="s")
    f = functools.partial(
        pl.kernel,
        mesh=mesh,
        compiler_params=pltpu.CompilerParams(use_tc_tiling_on_sc=False,
                                             needs_layout_passes=False),
        out_type=(
            jax.ShapeDtypeStruct((BATCH,), jnp.float32),
            jax.ShapeDtypeStruct((_NW, _LANES), jnp.float32),
        ),
        scratch_types=[
            pltpu.VMEM((_BPW, 3), jnp.int32),
            pltpu.VMEM((_BPW,), jnp.int32),
            pltpu.VMEM((_BPW,), jnp.int32),
            pltpu.VMEM((_BPW,), jnp.int32),
            pltpu.VMEM((_BPW, HIDDEN), jnp.float32),
            pltpu.VMEM((_BPW, HIDDEN), jnp.float32),
            pltpu.VMEM((_BPW, HIDDEN), jnp.float32),
            pltpu.VMEM((_BPW,), jnp.float32),
            pltpu.VMEM((_BPW,), jnp.float32),
            pltpu.VMEM((_BPW,), jnp.float32),
            pltpu.VMEM((_LANES,), jnp.float32),
            pltpu.SemaphoreType.DMA,
        ],
    )(_sc_body)
    return f(trip, user_embedding, item_embedding, item_bias)


def _loss_body(x_ref, l2_ref, out_ref):
    x = x_ref[...]
    ls = jnp.minimum(x, 0.0) - jnp.log(1.0 + jnp.exp(-jnp.abs(x)))
    l2 = jnp.sum(l2_ref[...])
    out_ref[0, 0] = 0.0001 * l2 - jnp.mean(ls)


def _tc_loss(x, l2p):
    return pl.pallas_call(
        _loss_body,
        out_shape=jax.ShapeDtypeStruct((1, 1), jnp.float32),
        out_specs=pl.BlockSpec(memory_space=pltpu.SMEM),
    )(x, l2p)


def kernel(input, user_embedding, item_embedding, item_bias):
    u_tail = user_embedding[_FULL_TILES * 128:].reshape(-1)
    i_tail = item_embedding[_FULL_TILES * 128:].reshape(-1)
    u_flat, i_flat = _detile(user_embedding.T, item_embedding.T,
                             u_tail, i_tail)
    x, l2p = _sc_call(input.astype(jnp.int32),
                      u_flat.reshape(NROWS, HIDDEN),
                      i_flat.reshape(NROWS, HIDDEN),
                      item_bias)
    return _tc_loss(x, l2p).reshape(())


# speedup vs baseline: 1.5032x; 1.5032x over previous
import functools

import jax
import jax.numpy as jnp
from jax import lax
from jax.experimental import pallas as pl
from jax.experimental.pallas import tpu as pltpu
from jax.experimental.pallas import tpu_sc as plsc

BATCH = 16384
HIDDEN = 32
NROWS = 1000000

_NC = 2
_NS = 16
_NW = _NC * _NS
_BPW = BATCH // _NW
_CHUNK = 128
_NCHUNK = _BPW // _CHUNK
_LANES = 16

_FULL_TILES = NROWS // 128
_TAIL = NROWS - _FULL_TILES * 128
_TPW = 245
_WIN = 4096


def _detile_body(uT, iT, u_tail, i_tail, u_flat, i_flat,
                 b0, b1, o0, o1, tailb, si0, si1, so0, so1):
    wid = lax.axis_index("s") * _NC + lax.axis_index("c")
    start = wid * _TPW
    lane = lax.iota(jnp.int32, _LANES)
    perm = [(lane + s) & 15 for s in range(_LANES)]
    permk32 = [lane * HIDDEN + p for p in perm]

    def tile_of(g):
        return jnp.minimum(start + g, _FULL_TILES - 1)

    def transpose_win(buf, out_v):
        def blk(kk):
            kvec = lane + kk * _LANES
            koff = kk * (_LANES * HIDDEN)
            for h0 in (0, 16):
                for s in range(_LANES):
                    hv = perm[s] + h0
                    sidx = permk32[s] + (koff + h0)
                    v = plsc.load_gather(buf, [hv, kvec])
                    plsc.store_scatter(out_v, [sidx], v)

        plsc.parallel_loop(0, 128 // _LANES, 1)(blk)

    for tT, flat in ((uT, u_flat), (iT, i_flat)):
        pltpu.async_copy(tT.at[:, pl.ds(tile_of(0) * 128, 128)], b0, si0)

        def pair(p, carry):
            g0 = 2 * p
            pltpu.make_async_copy(tT.at[:, pl.ds(0, 128)], b0, si0).wait()
            pltpu.async_copy(tT.at[:, pl.ds(tile_of(g0 + 1) * 128, 128)], b1, si1)
            transpose_win(b0, o0)
            pltpu.async_copy(o0, flat.at[pl.ds(tile_of(g0) * _WIN, _WIN)], so0)
            pltpu.make_async_copy(tT.at[:, pl.ds(0, 128)], b1, si1).wait()
            pltpu.async_copy(tT.at[:, pl.ds(tile_of(g0 + 2) * 128, 128)], b0, si0)
            transpose_win(b1, o1)
            pltpu.make_async_copy(flat.at[pl.ds(0, _WIN)], o0, so0).wait()
            pltpu.async_copy(o1, flat.at[pl.ds(tile_of(g0 + 1) * _WIN, _WIN)], so1)
            pltpu.make_async_copy(flat.at[pl.ds(0, _WIN)], o1, so1).wait()
            return carry

        lax.fori_loop(0, _TPW // 2 + 1, pair, 0)
        pltpu.make_async_copy(tT.at[:, pl.ds(0, 128)], b0, si0).wait()

    @pl.when(wid == 0)
    def _():
        pltpu.sync_copy(u_tail, tailb)
        pltpu.sync_copy(tailb,
                        u_flat.at[pl.ds(_FULL_TILES * _WIN, _TAIL * HIDDEN)])

    @pl.when(wid == 1)
    def _():
        pltpu.sync_copy(i_tail, tailb)
        pltpu.sync_copy(tailb,
                        i_flat.at[pl.ds(_FULL_TILES * _WIN, _TAIL * HIDDEN)])


def _detile(uT, iT, u_tail, i_tail):
    mesh = plsc.VectorSubcoreMesh(core_axis_name="c", subcore_axis_name="s")
    f = functools.partial(
        pl.kernel,
        mesh=mesh,
        compiler_params=pltpu.CompilerParams(use_tc_tiling_on_sc=True,
                                             needs_layout_passes=False),
        out_type=(
            jax.ShapeDtypeStruct((NROWS * HIDDEN,), jnp.float32),
            jax.ShapeDtypeStruct((NROWS * HIDDEN,), jnp.float32),
        ),
        scratch_types=[
            pltpu.VMEM((HIDDEN, 128), jnp.float32),
            pltpu.VMEM((HIDDEN, 128), jnp.float32),
            pltpu.VMEM((_WIN,), jnp.float32),
            pltpu.VMEM((_WIN,), jnp.float32),
            pltpu.VMEM((_TAIL * HIDDEN,), jnp.float32),
            pltpu.SemaphoreType.DMA,
            pltpu.SemaphoreType.DMA,
            pltpu.SemaphoreType.DMA,
            pltpu.SemaphoreType.DMA,
        ],
    )(_detile_body)
    return f(uT, iT, u_tail, i_tail)


def _sc_body(trip_hbm, uemb, iemb, ibias,
             x_out, l2_out,
             trip_v, idxu_v, idxi_v, idxj_v,
             u_v, i_v, j_v, ib_v, jb_v, x_v, l2_v, sem):
    wid = lax.axis_index("s") * _NC + lax.axis_index("c")
    base = wid * _BPW

    pltpu.sync_copy(trip_hbm.at[pl.ds(base, _BPW)], trip_v)

    lane = lax.iota(jnp.int32, _LANES)
    for k in range(_BPW // _LANES):
        rows = lane + (k * _LANES)
        dst = pl.ds(k * _LANES, _LANES)
        idxu_v[dst] = plsc.load_gather(trip_v, [rows, jnp.zeros((_LANES,), jnp.int32)])
        idxi_v[dst] = plsc.load_gather(trip_v, [rows, jnp.full((_LANES,), 1, jnp.int32)])
        idxj_v[dst] = plsc.load_gather(trip_v, [rows, jnp.full((_LANES,), 2, jnp.int32)])

    copies = []
    for k in range(_NCHUNK):
        s = pl.ds(k * _CHUNK, _CHUNK)
        copies.append(pltpu.async_copy(uemb.at[idxu_v.at[s]], u_v.at[s], sem))
        copies.append(pltpu.async_copy(iemb.at[idxi_v.at[s]], i_v.at[s], sem))
        copies.append(pltpu.async_copy(iemb.at[idxj_v.at[s]], j_v.at[s], sem))
        copies.append(pltpu.async_copy(ibias.at[idxi_v.at[s]], ib_v.at[s], sem))
        copies.append(pltpu.async_copy(ibias.at[idxj_v.at[s]], jb_v.at[s], sem))
    for c in copies:
        c.wait()

    def step(k, l2acc):
        off = k * _LANES
        rows = lane + off
        acc = ib_v[pl.ds(off, _LANES)] - jb_v[pl.ds(off, _LANES)]
        for h in range(HIDDEN):
            hv = jnp.full((_LANES,), h, jnp.int32)
            uh = plsc.load_gather(u_v, [rows, hv])
            ih = plsc.load_gather(i_v, [rows, hv])
            jh = plsc.load_gather(j_v, [rows, hv])
            acc = acc + uh * (ih - jh)
            l2acc = l2acc + (uh * uh + ih * ih + jh * jh)
        x_v[pl.ds(off, _LANES)] = acc
        return l2acc

    l2acc = lax.fori_loop(0, _BPW // _LANES, step, jnp.zeros((_LANES,), jnp.float32))
    l2_v[...] = l2acc

    pltpu.sync_copy(x_v, x_out.at[pl.ds(base, _BPW)])
    pltpu.sync_copy(l2_v, l2_out.at[wid])


def _sc_call(trip, user_embedding, item_embedding, item_bias):
    mesh = plsc.VectorSubcoreMesh(core_axis_name="c", subcore_axis_name="s")
    f = functools.partial(
        pl.kernel,
        mesh=mesh,
        compiler_params=pltpu.CompilerParams(use_tc_tiling_on_sc=False,
                                             needs_layout_passes=False),
        out_type=(
            jax.ShapeDtypeStruct((BATCH,), jnp.float32),
            jax.ShapeDtypeStruct((_NW, _LANES), jnp.float32),
        ),
        scratch_types=[
            pltpu.VMEM((_BPW, 3), jnp.int32),
            pltpu.VMEM((_BPW,), jnp.int32),
            pltpu.VMEM((_BPW,), jnp.int32),
            pltpu.VMEM((_BPW,), jnp.int32),
            pltpu.VMEM((_BPW, HIDDEN), jnp.float32),
            pltpu.VMEM((_BPW, HIDDEN), jnp.float32),
            pltpu.VMEM((_BPW, HIDDEN), jnp.float32),
            pltpu.VMEM((_BPW,), jnp.float32),
            pltpu.VMEM((_BPW,), jnp.float32),
            pltpu.VMEM((_BPW,), jnp.float32),
            pltpu.VMEM((_LANES,), jnp.float32),
            pltpu.SemaphoreType.DMA,
        ],
    )(_sc_body)
    return f(trip, user_embedding, item_embedding, item_bias)


def _loss_body(x_ref, l2_ref, out_ref):
    x = x_ref[...]
    ls = jnp.minimum(x, 0.0) - jnp.log(1.0 + jnp.exp(-jnp.abs(x)))
    l2 = jnp.sum(l2_ref[...])
    out_ref[0, 0] = 0.0001 * l2 - jnp.mean(ls)


def _tc_loss(x, l2p):
    return pl.pallas_call(
        _loss_body,
        out_shape=jax.ShapeDtypeStruct((1, 1), jnp.float32),
        out_specs=pl.BlockSpec(memory_space=pltpu.SMEM),
    )(x, l2p)


def kernel(input, user_embedding, item_embedding, item_bias):
    u_tail = user_embedding[_FULL_TILES * 128:].reshape(-1)
    i_tail = item_embedding[_FULL_TILES * 128:].reshape(-1)
    u_flat, i_flat = _detile(user_embedding.T, item_embedding.T,
                             u_tail, i_tail)
    x, l2p = _sc_call(input.astype(jnp.int32),
                      u_flat.reshape(NROWS, HIDDEN),
                      i_flat.reshape(NROWS, HIDDEN),
                      item_bias)
    return _tc_loss(x, l2p).reshape(())


# speedup vs baseline: 2.7357x; 1.8199x over previous
import functools

import jax
import jax.numpy as jnp
from jax import lax
from jax.experimental import pallas as pl
from jax.experimental.pallas import tpu as pltpu
from jax.experimental.pallas import tpu_sc as plsc

BATCH = 16384
HIDDEN = 32
NROWS = 1000000

_NC = 2
_NS = 16
_NW = _NC * _NS
_BPW = BATCH // _NW
_CHUNK = 128
_NCHUNK = _BPW // _CHUNK
_LANES = 16

_WL = 512
_FULL_TILES = NROWS // _WL
_TAIL = NROWS - _FULL_TILES * _WL
_TPW = 62
_WIN = _WL * HIDDEN


def _detile_body(uT, iT, u_tail, i_tail, u_flat, i_flat,
                 b0, b1, o0, o1, tailb, si0, si1, so0, so1):
    wid = lax.axis_index("s") * _NC + lax.axis_index("c")
    start = wid * _TPW
    lane = lax.iota(jnp.int32, _LANES)
    perm = [(lane + s) & 15 for s in range(_LANES)]
    permk32 = [lane * HIDDEN + p for p in perm]

    def tile_of(g):
        return jnp.minimum(start + g, _FULL_TILES - 1)

    def transpose_win(buf, out_v):
        def blk(kk):
            kvec = lane + kk * _LANES
            koff = kk * (_LANES * HIDDEN)
            for h0 in (0, 16):
                for s in range(_LANES):
                    hv = perm[s] + h0
                    sidx = permk32[s] + (koff + h0)
                    v = plsc.load_gather(buf, [hv, kvec])
                    plsc.store_scatter(out_v, [sidx], v)

        plsc.parallel_loop(0, _WL // _LANES, 1)(blk)

    for tT, flat in ((uT, u_flat), (iT, i_flat)):
        pltpu.async_copy(tT.at[:, pl.ds(tile_of(0) * _WL, _WL)], b0, si0)

        def pair(p, carry):
            g0 = 2 * p
            pltpu.make_async_copy(tT.at[:, pl.ds(0, _WL)], b0, si0).wait()
            pltpu.async_copy(tT.at[:, pl.ds(tile_of(g0 + 1) * _WL, _WL)], b1, si1)
            transpose_win(b0, o0)
            pltpu.async_copy(o0, flat.at[pl.ds(tile_of(g0) * _WIN, _WIN)], so0)
            pltpu.make_async_copy(tT.at[:, pl.ds(0, _WL)], b1, si1).wait()
            pltpu.async_copy(tT.at[:, pl.ds(tile_of(g0 + 2) * _WL, _WL)], b0, si0)
            transpose_win(b1, o1)
            pltpu.make_async_copy(flat.at[pl.ds(0, _WIN)], o0, so0).wait()
            pltpu.async_copy(o1, flat.at[pl.ds(tile_of(g0 + 1) * _WIN, _WIN)], so1)
            pltpu.make_async_copy(flat.at[pl.ds(0, _WIN)], o1, so1).wait()
            return carry

        lax.fori_loop(0, _TPW // 2 + 1, pair, 0)
        pltpu.make_async_copy(tT.at[:, pl.ds(0, _WL)], b0, si0).wait()

    @pl.when(wid == 0)
    def _():
        pltpu.sync_copy(u_tail, tailb)
        pltpu.sync_copy(tailb,
                        u_flat.at[pl.ds(_FULL_TILES * _WIN, _TAIL * HIDDEN)])

    @pl.when(wid == 1)
    def _():
        pltpu.sync_copy(i_tail, tailb)
        pltpu.sync_copy(tailb,
                        i_flat.at[pl.ds(_FULL_TILES * _WIN, _TAIL * HIDDEN)])


def _detile(uT, iT, u_tail, i_tail):
    mesh = plsc.VectorSubcoreMesh(core_axis_name="c", subcore_axis_name="s")
    f = functools.partial(
        pl.kernel,
        mesh=mesh,
        compiler_params=pltpu.CompilerParams(use_tc_tiling_on_sc=True,
                                             needs_layout_passes=False),
        out_type=(
            jax.ShapeDtypeStruct((NROWS * HIDDEN,), jnp.float32),
            jax.ShapeDtypeStruct((NROWS * HIDDEN,), jnp.float32),
        ),
        scratch_types=[
            pltpu.VMEM((HIDDEN, _WL), jnp.float32),
            pltpu.VMEM((HIDDEN, _WL), jnp.float32),
            pltpu.VMEM((_WIN,), jnp.float32),
            pltpu.VMEM((_WIN,), jnp.float32),
            pltpu.VMEM((_TAIL * HIDDEN,), jnp.float32),
            pltpu.SemaphoreType.DMA,
            pltpu.SemaphoreType.DMA,
            pltpu.SemaphoreType.DMA,
            pltpu.SemaphoreType.DMA,
        ],
    )(_detile_body)
    return f(uT, iT, u_tail, i_tail)


def _sc_body(trip_hbm, uemb, iemb, ibias,
             x_out, l2_out,
             trip_v, idxu_v, idxi_v, idxj_v,
             u_v, i_v, j_v, ib_v, jb_v, x_v, l2_v, sem):
    wid = lax.axis_index("s") * _NC + lax.axis_index("c")
    base = wid * _BPW

    pltpu.sync_copy(trip_hbm.at[pl.ds(base, _BPW)], trip_v)

    lane = lax.iota(jnp.int32, _LANES)
    for k in range(_BPW // _LANES):
        rows = lane + (k * _LANES)
        dst = pl.ds(k * _LANES, _LANES)
        idxu_v[dst] = plsc.load_gather(trip_v, [rows, jnp.zeros((_LANES,), jnp.int32)])
        idxi_v[dst] = plsc.load_gather(trip_v, [rows, jnp.full((_LANES,), 1, jnp.int32)])
        idxj_v[dst] = plsc.load_gather(trip_v, [rows, jnp.full((_LANES,), 2, jnp.int32)])

    copies = []
    for k in range(_NCHUNK):
        s = pl.ds(k * _CHUNK, _CHUNK)
        copies.append(pltpu.async_copy(uemb.at[idxu_v.at[s]], u_v.at[s], sem))
        copies.append(pltpu.async_copy(iemb.at[idxi_v.at[s]], i_v.at[s], sem))
        copies.append(pltpu.async_copy(iemb.at[idxj_v.at[s]], j_v.at[s], sem))
        copies.append(pltpu.async_copy(ibias.at[idxi_v.at[s]], ib_v.at[s], sem))
        copies.append(pltpu.async_copy(ibias.at[idxj_v.at[s]], jb_v.at[s], sem))
    for c in copies:
        c.wait()

    def step(k, l2acc):
        off = k * _LANES
        rows = lane + off
        acc = ib_v[pl.ds(off, _LANES)] - jb_v[pl.ds(off, _LANES)]
        for h in range(HIDDEN):
            hv = jnp.full((_LANES,), h, jnp.int32)
            uh = plsc.load_gather(u_v, [rows, hv])
            ih = plsc.load_gather(i_v, [rows, hv])
            jh = plsc.load_gather(j_v, [rows, hv])
            acc = acc + uh * (ih - jh)
            l2acc = l2acc + (uh * uh + ih * ih + jh * jh)
        x_v[pl.ds(off, _LANES)] = acc
        return l2acc

    l2acc = lax.fori_loop(0, _BPW // _LANES, step, jnp.zeros((_LANES,), jnp.float32))
    l2_v[...] = l2acc

    pltpu.sync_copy(x_v, x_out.at[pl.ds(base, _BPW)])
    pltpu.sync_copy(l2_v, l2_out.at[wid])


def _sc_call(trip, user_embedding, item_embedding, item_bias):
    mesh = plsc.VectorSubcoreMesh(core_axis_name="c", subcore_axis_name="s")
    f = functools.partial(
        pl.kernel,
        mesh=mesh,
        compiler_params=pltpu.CompilerParams(use_tc_tiling_on_sc=False,
                                             needs_layout_passes=False),
        out_type=(
            jax.ShapeDtypeStruct((BATCH,), jnp.float32),
            jax.ShapeDtypeStruct((_NW, _LANES), jnp.float32),
        ),
        scratch_types=[
            pltpu.VMEM((_BPW, 3), jnp.int32),
            pltpu.VMEM((_BPW,), jnp.int32),
            pltpu.VMEM((_BPW,), jnp.int32),
            pltpu.VMEM((_BPW,), jnp.int32),
            pltpu.VMEM((_BPW, HIDDEN), jnp.float32),
            pltpu.VMEM((_BPW, HIDDEN), jnp.float32),
            pltpu.VMEM((_BPW, HIDDEN), jnp.float32),
            pltpu.VMEM((_BPW,), jnp.float32),
            pltpu.VMEM((_BPW,), jnp.float32),
            pltpu.VMEM((_BPW,), jnp.float32),
            pltpu.VMEM((_LANES,), jnp.float32),
            pltpu.SemaphoreType.DMA,
        ],
    )(_sc_body)
    return f(trip, user_embedding, item_embedding, item_bias)


def _loss_body(x_ref, l2_ref, out_ref):
    x = x_ref[...]
    ls = jnp.minimum(x, 0.0) - jnp.log(1.0 + jnp.exp(-jnp.abs(x)))
    l2 = jnp.sum(l2_ref[...])
    out_ref[0, 0] = 0.0001 * l2 - jnp.mean(ls)


def _tc_loss(x, l2p):
    return pl.pallas_call(
        _loss_body,
        out_shape=jax.ShapeDtypeStruct((1, 1), jnp.float32),
        out_specs=pl.BlockSpec(memory_space=pltpu.SMEM),
    )(x, l2p)


def kernel(input, user_embedding, item_embedding, item_bias):
    u_tail = user_embedding[_FULL_TILES * _WL:].reshape(-1)
    i_tail = item_embedding[_FULL_TILES * _WL:].reshape(-1)
    u_flat, i_flat = _detile(user_embedding.T, item_embedding.T,
                             u_tail, i_tail)
    x, l2p = _sc_call(input.astype(jnp.int32),
                      u_flat.reshape(NROWS, HIDDEN),
                      i_flat.reshape(NROWS, HIDDEN),
                      item_bias)
    return _tc_loss(x, l2p).reshape(())


# speedup vs baseline: 2.9553x; 1.0802x over previous
import functools

import jax
import jax.numpy as jnp
from jax import lax
from jax.experimental import pallas as pl
from jax.experimental.pallas import tpu as pltpu
from jax.experimental.pallas import tpu_sc as plsc

BATCH = 16384
HIDDEN = 32
NROWS = 1000000

_NC = 2
_NS = 16
_NW = _NC * _NS
_BPW = BATCH // _NW
_CHUNK = 128
_NCHUNK = _BPW // _CHUNK
_LANES = 16

_WL = 768
_FULL_TILES = NROWS // _WL
_TAIL = NROWS - _FULL_TILES * _WL
_TPW = 41
_WIN = _WL * HIDDEN


def _detile_body(uT, iT, u_tail, i_tail, u_flat, i_flat,
                 b0, b1, o0, o1, tailb, si0, si1, so0, so1):
    wid = lax.axis_index("s") * _NC + lax.axis_index("c")
    start = wid * _TPW
    lane = lax.iota(jnp.int32, _LANES)
    perm = [(lane + s) & 15 for s in range(_LANES)]
    permk32 = [lane * HIDDEN + p for p in perm]

    def tile_of(g):
        return jnp.minimum(start + g, _FULL_TILES - 1)

    def transpose_win(buf, out_v):
        def blk(kk):
            kvec = lane + kk * _LANES
            koff = kk * (_LANES * HIDDEN)
            for h0 in (0, 16):
                for s in range(_LANES):
                    hv = perm[s] + h0
                    sidx = permk32[s] + (koff + h0)
                    v = plsc.load_gather(buf, [hv, kvec])
                    plsc.store_scatter(out_v, [sidx], v)

        plsc.parallel_loop(0, _WL // _LANES, 1)(blk)

    for tT, flat in ((uT, u_flat), (iT, i_flat)):
        pltpu.async_copy(tT.at[:, pl.ds(tile_of(0) * _WL, _WL)], b0, si0)

        def pair(p, carry):
            g0 = 2 * p
            pltpu.make_async_copy(tT.at[:, pl.ds(0, _WL)], b0, si0).wait()
            pltpu.async_copy(tT.at[:, pl.ds(tile_of(g0 + 1) * _WL, _WL)], b1, si1)
            transpose_win(b0, o0)
            pltpu.async_copy(o0, flat.at[pl.ds(tile_of(g0) * _WIN, _WIN)], so0)
            pltpu.make_async_copy(tT.at[:, pl.ds(0, _WL)], b1, si1).wait()
            pltpu.async_copy(tT.at[:, pl.ds(tile_of(g0 + 2) * _WL, _WL)], b0, si0)
            transpose_win(b1, o1)
            pltpu.make_async_copy(flat.at[pl.ds(0, _WIN)], o0, so0).wait()
            pltpu.async_copy(o1, flat.at[pl.ds(tile_of(g0 + 1) * _WIN, _WIN)], so1)
            pltpu.make_async_copy(flat.at[pl.ds(0, _WIN)], o1, so1).wait()
            return carry

        lax.fori_loop(0, _TPW // 2 + 1, pair, 0)
        pltpu.make_async_copy(tT.at[:, pl.ds(0, _WL)], b0, si0).wait()

    @pl.when(wid == 0)
    def _():
        pltpu.sync_copy(u_tail, tailb)
        pltpu.sync_copy(tailb,
                        u_flat.at[pl.ds(_FULL_TILES * _WIN, _TAIL * HIDDEN)])

    @pl.when(wid == 1)
    def _():
        pltpu.sync_copy(i_tail, tailb)
        pltpu.sync_copy(tailb,
                        i_flat.at[pl.ds(_FULL_TILES * _WIN, _TAIL * HIDDEN)])


def _detile(uT, iT, u_tail, i_tail):
    mesh = plsc.VectorSubcoreMesh(core_axis_name="c", subcore_axis_name="s")
    f = functools.partial(
        pl.kernel,
        mesh=mesh,
        compiler_params=pltpu.CompilerParams(use_tc_tiling_on_sc=True,
                                             needs_layout_passes=False),
        out_type=(
            jax.ShapeDtypeStruct((NROWS * HIDDEN,), jnp.float32),
            jax.ShapeDtypeStruct((NROWS * HIDDEN,), jnp.float32),
        ),
        scratch_types=[
            pltpu.VMEM((HIDDEN, _WL), jnp.float32),
            pltpu.VMEM((HIDDEN, _WL), jnp.float32),
            pltpu.VMEM((_WIN,), jnp.float32),
            pltpu.VMEM((_WIN,), jnp.float32),
            pltpu.VMEM((_TAIL * HIDDEN,), jnp.float32),
            pltpu.SemaphoreType.DMA,
            pltpu.SemaphoreType.DMA,
            pltpu.SemaphoreType.DMA,
            pltpu.SemaphoreType.DMA,
        ],
    )(_detile_body)
    return f(uT, iT, u_tail, i_tail)


def _sc_body(trip_hbm, uemb, iemb, ibias,
             x_out, l2_out,
             trip_v, idxu_v, idxi_v, idxj_v,
             u_v, i_v, j_v, ib_v, jb_v, x_v, l2_v, sem):
    wid = lax.axis_index("s") * _NC + lax.axis_index("c")
    base = wid * _BPW

    pltpu.sync_copy(trip_hbm.at[pl.ds(base, _BPW)], trip_v)

    lane = lax.iota(jnp.int32, _LANES)
    for k in range(_BPW // _LANES):
        rows = lane + (k * _LANES)
        dst = pl.ds(k * _LANES, _LANES)
        idxu_v[dst] = plsc.load_gather(trip_v, [rows, jnp.zeros((_LANES,), jnp.int32)])
        idxi_v[dst] = plsc.load_gather(trip_v, [rows, jnp.full((_LANES,), 1, jnp.int32)])
        idxj_v[dst] = plsc.load_gather(trip_v, [rows, jnp.full((_LANES,), 2, jnp.int32)])

    copies = []
    for k in range(_NCHUNK):
        s = pl.ds(k * _CHUNK, _CHUNK)
        copies.append(pltpu.async_copy(uemb.at[idxu_v.at[s]], u_v.at[s], sem))
        copies.append(pltpu.async_copy(iemb.at[idxi_v.at[s]], i_v.at[s], sem))
        copies.append(pltpu.async_copy(iemb.at[idxj_v.at[s]], j_v.at[s], sem))
        copies.append(pltpu.async_copy(ibias.at[idxi_v.at[s]], ib_v.at[s], sem))
        copies.append(pltpu.async_copy(ibias.at[idxj_v.at[s]], jb_v.at[s], sem))
    for c in copies:
        c.wait()

    def step(k, l2acc):
        off = k * _LANES
        rows = lane + off
        acc = ib_v[pl.ds(off, _LANES)] - jb_v[pl.ds(off, _LANES)]
        for h in range(HIDDEN):
            hv = jnp.full((_LANES,), h, jnp.int32)
            uh = plsc.load_gather(u_v, [rows, hv])
            ih = plsc.load_gather(i_v, [rows, hv])
            jh = plsc.load_gather(j_v, [rows, hv])
            acc = acc + uh * (ih - jh)
            l2acc = l2acc + (uh * uh + ih * ih + jh * jh)
        x_v[pl.ds(off, _LANES)] = acc
        return l2acc

    l2acc = lax.fori_loop(0, _BPW // _LANES, step, jnp.zeros((_LANES,), jnp.float32))
    l2_v[...] = l2acc

    pltpu.sync_copy(x_v, x_out.at[pl.ds(base, _BPW)])
    pltpu.sync_copy(l2_v, l2_out.at[wid])


def _sc_call(trip, user_embedding, item_embedding, item_bias):
    mesh = plsc.VectorSubcoreMesh(core_axis_name="c", subcore_axis_name="s")
    f = functools.partial(
        pl.kernel,
        mesh=mesh,
        compiler_params=pltpu.CompilerParams(use_tc_tiling_on_sc=False,
                                             needs_layout_passes=False),
        out_type=(
            jax.ShapeDtypeStruct((BATCH,), jnp.float32),
            jax.ShapeDtypeStruct((_NW, _LANES), jnp.float32),
        ),
        scratch_types=[
            pltpu.VMEM((_BPW, 3), jnp.int32),
            pltpu.VMEM((_BPW,), jnp.int32),
            pltpu.VMEM((_BPW,), jnp.int32),
            pltpu.VMEM((_BPW,), jnp.int32),
            pltpu.VMEM((_BPW, HIDDEN), jnp.float32),
            pltpu.VMEM((_BPW, HIDDEN), jnp.float32),
            pltpu.VMEM((_BPW, HIDDEN), jnp.float32),
            pltpu.VMEM((_BPW,), jnp.float32),
            pltpu.VMEM((_BPW,), jnp.float32),
            pltpu.VMEM((_BPW,), jnp.float32),
            pltpu.VMEM((_LANES,), jnp.float32),
            pltpu.SemaphoreType.DMA,
        ],
    )(_sc_body)
    return f(trip, user_embedding, item_embedding, item_bias)


def _loss_body(x_ref, l2_ref, out_ref):
    x = x_ref[...]
    ls = jnp.minimum(x, 0.0) - jnp.log(1.0 + jnp.exp(-jnp.abs(x)))
    l2 = jnp.sum(l2_ref[...])
    out_ref[0, 0] = 0.0001 * l2 - jnp.mean(ls)


def _tc_loss(x, l2p):
    return pl.pallas_call(
        _loss_body,
        out_shape=jax.ShapeDtypeStruct((1, 1), jnp.float32),
        out_specs=pl.BlockSpec(memory_space=pltpu.SMEM),
    )(x, l2p)


def kernel(input, user_embedding, item_embedding, item_bias):
    u_tail = user_embedding[_FULL_TILES * _WL:].reshape(-1)
    i_tail = item_embedding[_FULL_TILES * _WL:].reshape(-1)
    u_flat, i_flat = _detile(user_embedding.T, item_embedding.T,
                             u_tail, i_tail)
    x, l2p = _sc_call(input.astype(jnp.int32),
                      u_flat.reshape(NROWS, HIDDEN),
                      i_flat.reshape(NROWS, HIDDEN),
                      item_bias)
    return _tc_loss(x, l2p).reshape(())
